# Initial kernel scaffold; baseline (speedup 1.0000x reference)
import jax
import jax.numpy as jnp
from jax.experimental import pallas as pl


def kernel(x, edge_index, Wk, bk, Wq, bq, Wv, bv, a_rel, m_rel, p_rel, Wa, ba, skip, Wn, bn, pos_table, ln_g, ln_b):
    raise NotImplementedError("write your pallas kernel here")


# speedup vs baseline: 19.7816x; 19.7816x over previous
import functools

import jax
import jax.numpy as jnp
from jax.experimental import pallas as pl
from jax.experimental.pallas import tpu as pltpu

_SQRT_2_OVER_PI = 0.7978845608028654


def _proj_body(x_ref, wk_ref, bk_ref, wq_ref, bq_ref, wv_ref, bv_ref,
               k_ref, q_ref, v_ref):
    xb = x_ref[...]
    k_ref[...] = jnp.dot(xb, wk_ref[...], preferred_element_type=jnp.float32) + bk_ref[...]
    q_ref[...] = jnp.dot(xb, wq_ref[...], preferred_element_type=jnp.float32) + bq_ref[...]
    v_ref[...] = jnp.dot(xb, wv_ref[...], preferred_element_type=jnp.float32) + bv_ref[...]


def _projections(x, Wk2, bk2, Wq, bq, Wv2, bv2, tile):
    n, d = x.shape
    grid = n // tile
    full = lambda i: (0, 0)
    row = lambda i: (i, 0)
    return pl.pallas_call(
        _proj_body,
        grid=(grid,),
        in_specs=[
            pl.BlockSpec((tile, d), row),
            pl.BlockSpec((d, d), full),
            pl.BlockSpec((1, d), full),
            pl.BlockSpec((d, d), full),
            pl.BlockSpec((1, d), full),
            pl.BlockSpec((d, d), full),
            pl.BlockSpec((1, d), full),
        ],
        out_specs=[
            pl.BlockSpec((tile, d), row),
            pl.BlockSpec((tile, d), row),
            pl.BlockSpec((tile, d), row),
        ],
        out_shape=[jax.ShapeDtypeStruct((n, d), jnp.float32)] * 3,
    )(x, Wk2, bk2, Wq, bq, Wv2, bv2)


def _post_body(num_ref, den_ref, x_ref, pos_ref, wa_ref, ba_ref, wn_ref,
               bn_ref, beta_ref, lng_ref, lnb_ref, o_ref):
    t = num_ref.shape[0]
    h = den_ref.shape[1]
    d = num_ref.shape[1]
    dh = d // h
    num = num_ref[...]
    den = den_ref[...]
    drep = jnp.reshape(jnp.broadcast_to(den[:, :, None], (t, h, dh)), (t, d))
    agg = num / (drep + 1e-16)
    g = 0.5 * agg * (1.0 + jnp.tanh(_SQRT_2_OVER_PI * (agg + 0.044715 * agg * agg * agg)))
    o = jnp.dot(g, wa_ref[...], preferred_element_type=jnp.float32) + ba_ref[...]
    beta = beta_ref[0, 0]
    o = beta * o + (1.0 - beta) * x_ref[...]
    hh = jnp.dot(o, wn_ref[...], preferred_element_type=jnp.float32) + bn_ref[...] + pos_ref[...]
    mu = jnp.mean(hh, axis=-1, keepdims=True)
    var = jnp.mean((hh - mu) ** 2, axis=-1, keepdims=True)
    o_ref[...] = (hh - mu) * jax.lax.rsqrt(var + 1e-12) * lng_ref[...] + lnb_ref[...]


def _post_stage(num, den, x, pos_table, Wa, ba, Wn, bn, beta, ln_g, ln_b, tile):
    n, d = x.shape
    h = den.shape[1]
    grid = n // tile
    full = lambda i: (0, 0)
    row = lambda i: (i, 0)
    return pl.pallas_call(
        _post_body,
        grid=(grid,),
        in_specs=[
            pl.BlockSpec((tile, d), row),
            pl.BlockSpec((tile, h), row),
            pl.BlockSpec((tile, d), row),
            pl.BlockSpec((tile, d), row),
            pl.BlockSpec((d, d), full),
            pl.BlockSpec((1, d), full),
            pl.BlockSpec((d, d), full),
            pl.BlockSpec((1, d), full),
            pl.BlockSpec((1, 1), full),
            pl.BlockSpec((1, d), full),
            pl.BlockSpec((1, d), full),
        ],
        out_specs=pl.BlockSpec((tile, d), row),
        out_shape=jax.ShapeDtypeStruct((n, d), jnp.float32),
    )(num, den, x, pos_table, Wa, ba, Wn, bn, beta, ln_g, ln_b)


def kernel(x, edge_index, Wk, bk, Wq, bq, Wv, bv, a_rel, m_rel, p_rel,
           Wa, ba, skip, Wn, bn, pos_table, ln_g, ln_b):
    n, d = x.shape
    heads, dh = a_rel.shape[0], a_rel.shape[1]
    e = edge_index.shape[1]

    scale = (p_rel / jnp.sqrt(jnp.float32(dh)))
    Wk2 = jnp.einsum('ihd,hde->ihe', Wk.reshape(d, heads, dh), a_rel)
    Wk2 = (Wk2 * scale[None, :, None]).reshape(d, d)
    bk2 = (jnp.einsum('hd,hde->he', bk.reshape(heads, dh), a_rel) * scale[:, None]).reshape(1, d)
    Wv2 = jnp.einsum('ihd,hde->ihe', Wv.reshape(d, heads, dh), m_rel).reshape(d, d)
    bv2 = jnp.einsum('hd,hde->he', bv.reshape(heads, dh), m_rel).reshape(1, d)

    k, q, v = _projections(x, Wk2, bk2, Wq, bq.reshape(1, d), Wv2, bv2, tile=2000)

    src = edge_index[0].astype(jnp.int32)
    dst = edge_index[1].astype(jnp.int32)
    kg = k[src]
    qg = q[dst]
    s = jnp.exp(jnp.sum((kg * qg).reshape(e, heads, dh), axis=-1))
    den = jax.ops.segment_sum(s, dst, num_segments=n)
    msg = v[src] * jnp.reshape(jnp.broadcast_to(s[:, :, None], (e, heads, dh)), (e, d))
    num = jax.ops.segment_sum(msg, dst, num_segments=n)

    beta = jax.nn.sigmoid(skip).reshape(1, 1)
    return _post_stage(num, den, x, pos_table, Wa, ba.reshape(1, d),
                       Wn, bn.reshape(1, d), beta, ln_g.reshape(1, d),
                       ln_b.reshape(1, d), tile=2000)
